# baseline (speedup 1.0000x reference)
import jax
import jax.numpy as jnp
from jax.experimental import pallas as pl
from jax.experimental.pallas import tpu as pltpu

B = 4096
D_IN = 1024
D_HID = 2048
D_EMB = 128

BM = 512


def _body(xq_ref, xd_ref, wq1_ref, bq1_ref, wq2_ref, bq2_ref,
          wd1_ref, bd1_ref, wd2_ref, bd2_ref, oq_ref, od_ref,
          wq1b, wq2b, wd1b, wd2b, xqb, xdb, hqb, hdb):
    @pl.when(pl.program_id(0) == 0)
    def _pack_weights():
        wq1b[:] = wq1_ref[:].astype(jnp.bfloat16)
        wq2b[:] = wq2_ref[:].astype(jnp.bfloat16)
        wd1b[:] = wd1_ref[:].astype(jnp.bfloat16)
        wd2b[:] = wd2_ref[:].astype(jnp.bfloat16)

    xqb[:] = xq_ref[:].astype(jnp.bfloat16)
    xdb[:] = xd_ref[:].astype(jnp.bfloat16)
    hq = jnp.maximum(
        jnp.dot(xqb[:], wq1b[:], preferred_element_type=jnp.float32)
        + bq1_ref[:], 0.0)
    hqb[:] = hq.astype(jnp.bfloat16)
    hd = jnp.maximum(
        jnp.dot(xdb[:], wd1b[:], preferred_element_type=jnp.float32)
        + bd1_ref[:], 0.0)
    hdb[:] = hd.astype(jnp.bfloat16)
    oq_ref[:] = (jnp.dot(hqb[:], wq2b[:], preferred_element_type=jnp.float32)
                 + bq2_ref[:])
    od_ref[:] = (jnp.dot(hdb[:], wd2b[:], preferred_element_type=jnp.float32)
                 + bd2_ref[:])


def kernel(query, document, Wq1, bq1, Wq2, bq2, Wd1, bd1, Wd2, bd2):
    bq1_2d = bq1.reshape(1, D_HID)
    bq2_2d = bq2.reshape(1, D_EMB)
    bd1_2d = bd1.reshape(1, D_HID)
    bd2_2d = bd2.reshape(1, D_EMB)

    x_spec = pl.BlockSpec((BM, D_IN), lambda i: (i, 0))
    w1_spec = pl.BlockSpec((D_IN, D_HID), lambda i: (0, 0))
    b1_spec = pl.BlockSpec((1, D_HID), lambda i: (0, 0))
    w2_spec = pl.BlockSpec((D_HID, D_EMB), lambda i: (0, 0))
    b2_spec = pl.BlockSpec((1, D_EMB), lambda i: (0, 0))
    o_spec = pl.BlockSpec((BM, D_EMB), lambda i: (i, 0))

    oq, od = pl.pallas_call(
        _body,
        grid=(B // BM,),
        in_specs=[x_spec, x_spec,
                  w1_spec, b1_spec, w2_spec, b2_spec,
                  w1_spec, b1_spec, w2_spec, b2_spec],
        out_specs=[o_spec, o_spec],
        out_shape=[jax.ShapeDtypeStruct((B, D_EMB), jnp.float32),
                   jax.ShapeDtypeStruct((B, D_EMB), jnp.float32)],
        scratch_shapes=[
            pltpu.VMEM((D_IN, D_HID), jnp.bfloat16),
            pltpu.VMEM((D_HID, D_EMB), jnp.bfloat16),
            pltpu.VMEM((D_IN, D_HID), jnp.bfloat16),
            pltpu.VMEM((D_HID, D_EMB), jnp.bfloat16),
            pltpu.VMEM((BM, D_IN), jnp.bfloat16),
            pltpu.VMEM((BM, D_IN), jnp.bfloat16),
            pltpu.VMEM((BM, D_HID), jnp.bfloat16),
            pltpu.VMEM((BM, D_HID), jnp.bfloat16),
        ],
        compiler_params=pltpu.CompilerParams(
            dimension_semantics=("arbitrary",),
        ),
    )(query, document, Wq1, bq1_2d, Wq2, bq2_2d, Wd1, bd1_2d, Wd2, bd2_2d)
    return (oq, od)
